# Initial kernel scaffold; baseline (speedup 1.0000x reference)
import functools
import jax
import jax.numpy as jnp
from jax import lax
from jax.experimental import pallas as pl
from jax.experimental.pallas import tpu as pltpu
from jax.experimental.pallas import tpu_sc as plsc

HID = 64
VOCAB = 64
ROWS = 4096
COLS = 200
B = ROWS * COLS
NC = 2
NS = 16
NW = NC * NS
ROWS_PER_W = B // NW
CHUNK = 128
NCHUNK = ROWS_PER_W // CHUNK


def _table_body(table_ref, w1_ref, b1_ref, w2_ref, b2_ref, gamma_ref,
                beta_ref, out_ref):
    h = table_ref[...]
    z = jnp.dot(h, w1_ref[...], preferred_element_type=jnp.float32) + b1_ref[...]
    z = jnp.maximum(z, 0.0)
    ff = jnp.dot(z, w2_ref[...], preferred_element_type=jnp.float32) + b2_ref[...]
    s = h + ff
    mu = jnp.mean(s, axis=-1, keepdims=True)
    var = jnp.mean((s - mu) * (s - mu), axis=-1, keepdims=True)
    shat = (s - mu) * lax.rsqrt(var + 1e-5)
    out_ref[...] = shat * gamma_ref[...] + beta_ref[...]


def _transform_table(table, w1, b1, w2, b2, gamma, beta):
    return pl.pallas_call(
        _table_body,
        out_shape=jax.ShapeDtypeStruct((VOCAB, HID), jnp.float32),
    )(table, w1, b1.reshape(1, -1), w2, b2.reshape(1, -1),
      gamma.reshape(1, -1), beta.reshape(1, -1))


_mesh = plsc.VectorSubcoreMesh(core_axis_name="c", subcore_axis_name="s")


@functools.partial(
    pl.kernel,
    out_type=jax.ShapeDtypeStruct((B, HID), jnp.float32),
    mesh=_mesh,
    scratch_types=[
        pltpu.VMEM((NCHUNK, CHUNK), jnp.int32),
        pltpu.VMEM((CHUNK, HID), jnp.float32),
        pltpu.SemaphoreType.DMA,
    ],
)
def _gather(x_hbm, tbl_hbm, out_hbm, idx_v, rows_v, gsem):
    wid = lax.axis_index("s") * NC + lax.axis_index("c")
    pltpu.sync_copy(x_hbm.at[wid], idx_v)
    base = wid * ROWS_PER_W

    def body(j, carry):
        pltpu.async_copy(tbl_hbm.at[idx_v.at[j]], rows_v, gsem).wait()
        pltpu.sync_copy(rows_v, out_hbm.at[pl.ds(base + j * CHUNK, CHUNK)])
        return carry

    lax.fori_loop(0, NCHUNK, body, 0)


def kernel(x, table, W1, b1, W2, b2, gamma, beta):
    out_table = _transform_table(table, W1, b1, W2, b2, gamma, beta)
    xr = x.reshape(NW, NCHUNK, CHUNK).astype(jnp.int32)
    flat = _gather(xr, out_table)
    return flat.reshape(ROWS, COLS, HID)


# speedup vs baseline: 2.4638x; 2.4638x over previous
import functools
import jax
import jax.numpy as jnp
from jax import lax
from jax.experimental import pallas as pl
from jax.experimental.pallas import tpu as pltpu
from jax.experimental.pallas import tpu_sc as plsc

HID = 64
VOCAB = 64
ROWS = 4096
COLS = 200
B = ROWS * COLS
NC = 2
NS = 16
NW = NC * NS
ROWS_PER_W = B // NW
CHUNK = 128
NCHUNK = ROWS_PER_W // CHUNK


def _table_body(table_ref, w1_ref, b1_ref, w2_ref, b2_ref, gamma_ref,
                beta_ref, out_ref):
    h = table_ref[...]
    z = jnp.dot(h, w1_ref[...], preferred_element_type=jnp.float32) + b1_ref[...]
    z = jnp.maximum(z, 0.0)
    ff = jnp.dot(z, w2_ref[...], preferred_element_type=jnp.float32) + b2_ref[...]
    s = h + ff
    mu = jnp.mean(s, axis=-1, keepdims=True)
    var = jnp.mean((s - mu) * (s - mu), axis=-1, keepdims=True)
    shat = (s - mu) * lax.rsqrt(var + 1e-5)
    out_ref[...] = shat * gamma_ref[...] + beta_ref[...]


def _transform_table(table, w1, b1, w2, b2, gamma, beta):
    return pl.pallas_call(
        _table_body,
        out_shape=jax.ShapeDtypeStruct((VOCAB, HID), jnp.float32),
    )(table, w1, b1.reshape(1, -1), w2, b2.reshape(1, -1),
      gamma.reshape(1, -1), beta.reshape(1, -1))


@functools.cache
def _make_gather():
    mesh = plsc.VectorSubcoreMesh(core_axis_name="c", subcore_axis_name="s")

    @functools.partial(
        pl.kernel,
        out_type=jax.ShapeDtypeStruct((B, HID), jnp.float32),
        mesh=mesh,
        scratch_types=[
            pltpu.VMEM((NCHUNK, CHUNK), jnp.int32),
            pltpu.VMEM((CHUNK, HID), jnp.float32),
            pltpu.SemaphoreType.DMA,
        ],
        compiler_params=pltpu.CompilerParams(use_tc_tiling_on_sc=False),
    )
    def _gather(x_hbm, tbl_hbm, out_hbm, idx_v, rows_v, gsem):
        wid = lax.axis_index("s") * NC + lax.axis_index("c")
        pltpu.sync_copy(x_hbm.at[wid], idx_v)
        base = wid * ROWS_PER_W

        def body(j, carry):
            pltpu.async_copy(tbl_hbm.at[idx_v.at[j]], rows_v, gsem).wait()
            pltpu.sync_copy(rows_v, out_hbm.at[pl.ds(base + j * CHUNK, CHUNK)])
            return carry

        lax.fori_loop(0, NCHUNK, body, 0)

    return _gather


def kernel(x, table, W1, b1, W2, b2, gamma, beta):
    out_table = _transform_table(table, W1, b1, W2, b2, gamma, beta)
    xr = x.reshape(NW, NCHUNK, CHUNK).astype(jnp.int32)
    flat = _make_gather()(xr, out_table)
    return flat.reshape(ROWS, COLS, HID)


# speedup vs baseline: 2.5084x; 1.0181x over previous
import functools
import jax
import jax.numpy as jnp
from jax import lax
from jax.experimental import pallas as pl
from jax.experimental.pallas import tpu as pltpu
from jax.experimental.pallas import tpu_sc as plsc

HID = 64
VOCAB = 64
ROWS = 4096
COLS = 200
B = ROWS * COLS
NC = 2
NS = 16
NW = NC * NS
ROWS_PER_W = B // NW
CHUNK = 128
NCHUNK = ROWS_PER_W // CHUNK


def _table_body(table_ref, w1_ref, b1_ref, w2_ref, b2_ref, gamma_ref,
                beta_ref, out_ref):
    h = table_ref[...]
    z = jnp.dot(h, w1_ref[...], preferred_element_type=jnp.float32) + b1_ref[...]
    z = jnp.maximum(z, 0.0)
    ff = jnp.dot(z, w2_ref[...], preferred_element_type=jnp.float32) + b2_ref[...]
    s = h + ff
    mu = jnp.mean(s, axis=-1, keepdims=True)
    var = jnp.mean((s - mu) * (s - mu), axis=-1, keepdims=True)
    shat = (s - mu) * lax.rsqrt(var + 1e-5)
    out_ref[...] = shat * gamma_ref[...] + beta_ref[...]


def _transform_table(table, w1, b1, w2, b2, gamma, beta):
    return pl.pallas_call(
        _table_body,
        out_shape=jax.ShapeDtypeStruct((VOCAB, HID), jnp.float32),
    )(table, w1, b1.reshape(1, -1), w2, b2.reshape(1, -1),
      gamma.reshape(1, -1), beta.reshape(1, -1))


NBUF = 4
DEPTH = 2


@functools.cache
def _make_gather():
    mesh = plsc.VectorSubcoreMesh(core_axis_name="c", subcore_axis_name="s")

    @functools.partial(
        pl.kernel,
        out_type=jax.ShapeDtypeStruct((B, HID), jnp.float32),
        mesh=mesh,
        scratch_types=[
            pltpu.VMEM((NCHUNK, CHUNK), jnp.int32),
            pltpu.VMEM((NBUF, CHUNK, HID), jnp.float32),
            pltpu.SemaphoreType.DMA,
            pltpu.SemaphoreType.DMA,
        ],
        compiler_params=pltpu.CompilerParams(use_tc_tiling_on_sc=False),
    )
    def _gather(x_hbm, tbl_hbm, out_hbm, idx_v, rows_v, gsem, ssem):
        wid = lax.axis_index("s") * NC + lax.axis_index("c")
        pltpu.sync_copy(x_hbm.at[wid], idx_v)
        base = wid * ROWS_PER_W

        def body(j, carry):
            @pl.when(j < NCHUNK)
            def _():
                b = j % NBUF

                @pl.when(j >= NBUF)
                def _():
                    pltpu.make_async_copy(
                        rows_v.at[b],
                        out_hbm.at[pl.ds(base + (j - NBUF) * CHUNK, CHUNK)],
                        ssem,
                    ).wait()

                pltpu.async_copy(tbl_hbm.at[idx_v.at[j]], rows_v.at[b], gsem)

            @pl.when(j >= DEPTH)
            def _():
                i = j - DEPTH
                bi = i % NBUF
                pltpu.make_async_copy(
                    tbl_hbm.at[idx_v.at[i]], rows_v.at[bi], gsem
                ).wait()
                pltpu.async_copy(
                    rows_v.at[bi],
                    out_hbm.at[pl.ds(base + i * CHUNK, CHUNK)],
                    ssem,
                )

            return carry

        lax.fori_loop(0, NCHUNK + DEPTH, body, 0)

        def drain(j, carry):
            pltpu.make_async_copy(
                rows_v.at[j % NBUF],
                out_hbm.at[pl.ds(base + j * CHUNK, CHUNK)],
                ssem,
            ).wait()
            return carry

        lax.fori_loop(NCHUNK - NBUF, NCHUNK, drain, 0)

    return _gather


def kernel(x, table, W1, b1, W2, b2, gamma, beta):
    out_table = _transform_table(table, W1, b1, W2, b2, gamma, beta)
    xr = x.reshape(NW, NCHUNK, CHUNK).astype(jnp.int32)
    flat = _make_gather()(xr, out_table)
    return flat.reshape(ROWS, COLS, HID)


# speedup vs baseline: 2.5124x; 1.0016x over previous
import functools
import jax
import jax.numpy as jnp
from jax import lax
from jax.experimental import pallas as pl
from jax.experimental.pallas import tpu as pltpu
from jax.experimental.pallas import tpu_sc as plsc

HID = 64
VOCAB = 64
ROWS = 4096
COLS = 200
B = ROWS * COLS
NC = 2
NS = 16
NW = NC * NS
ROWS_PER_W = B // NW
CHUNK = 128
NCHUNK = ROWS_PER_W // CHUNK


def _table_body(table_ref, w1_ref, b1_ref, w2_ref, b2_ref, gamma_ref,
                beta_ref, out_ref):
    h = table_ref[...]
    z = jnp.dot(h, w1_ref[...], preferred_element_type=jnp.float32) + b1_ref[...]
    z = jnp.maximum(z, 0.0)
    ff = jnp.dot(z, w2_ref[...], preferred_element_type=jnp.float32) + b2_ref[...]
    s = h + ff
    mu = jnp.mean(s, axis=-1, keepdims=True)
    var = jnp.mean((s - mu) * (s - mu), axis=-1, keepdims=True)
    shat = (s - mu) * lax.rsqrt(var + 1e-5)
    out_ref[...] = shat * gamma_ref[...] + beta_ref[...]


def _transform_table(table, w1, b1, w2, b2, gamma, beta):
    return pl.pallas_call(
        _table_body,
        out_shape=jax.ShapeDtypeStruct((VOCAB, HID), jnp.float32),
    )(table, w1, b1.reshape(1, -1), w2, b2.reshape(1, -1),
      gamma.reshape(1, -1), beta.reshape(1, -1))


NBUF = 4
DEPTH = 2
RPW = ROWS // NW
G0, G1 = 128, COLS - 128


@functools.cache
def _make_gather():
    mesh = plsc.VectorSubcoreMesh(core_axis_name="c", subcore_axis_name="s")

    @functools.partial(
        pl.kernel,
        out_type=jax.ShapeDtypeStruct((ROWS, COLS, HID), jnp.float32),
        mesh=mesh,
        scratch_types=[
            pltpu.VMEM((ROWS_PER_W,), jnp.int32),
            pltpu.VMEM((NBUF, COLS, HID), jnp.float32),
            pltpu.SemaphoreType.DMA,
            pltpu.SemaphoreType.DMA,
        ],
        compiler_params=pltpu.CompilerParams(use_tc_tiling_on_sc=False),
    )
    def _gather(x_hbm, tbl_hbm, out_hbm, idx_v, rows_v, gsem, ssem):
        wid = lax.axis_index("s") * NC + lax.axis_index("c")
        pltpu.sync_copy(x_hbm.at[wid], idx_v)
        row0 = wid * RPW

        def issue_gathers(r, b):
            pltpu.async_copy(
                tbl_hbm.at[idx_v.at[pl.ds(r * COLS, G0)]],
                rows_v.at[b, pl.ds(0, G0)], gsem)
            pltpu.async_copy(
                tbl_hbm.at[idx_v.at[pl.ds(r * COLS + G0, G1)]],
                rows_v.at[b, pl.ds(G0, G1)], gsem)

        def wait_gathers(r, b):
            pltpu.make_async_copy(
                tbl_hbm.at[idx_v.at[pl.ds(r * COLS, G0)]],
                rows_v.at[b, pl.ds(0, G0)], gsem).wait()
            pltpu.make_async_copy(
                tbl_hbm.at[idx_v.at[pl.ds(r * COLS + G0, G1)]],
                rows_v.at[b, pl.ds(G0, G1)], gsem).wait()

        def body(j, carry):
            @pl.when(j < RPW)
            def _():
                b = j % NBUF

                @pl.when(j >= NBUF)
                def _():
                    pltpu.make_async_copy(
                        rows_v.at[b], out_hbm.at[row0 + j - NBUF], ssem
                    ).wait()

                issue_gathers(j, b)

            @pl.when(j >= DEPTH)
            def _():
                i = j - DEPTH
                bi = i % NBUF
                wait_gathers(i, bi)
                pltpu.async_copy(rows_v.at[bi], out_hbm.at[row0 + i], ssem)

            return carry

        lax.fori_loop(0, RPW + DEPTH, body, 0)

        def drain(j, carry):
            pltpu.make_async_copy(
                rows_v.at[j % NBUF], out_hbm.at[row0 + j], ssem
            ).wait()
            return carry

        lax.fori_loop(RPW - NBUF, RPW, drain, 0)

    return _gather


def kernel(x, table, W1, b1, W2, b2, gamma, beta):
    out_table = _transform_table(table, W1, b1, W2, b2, gamma, beta)
    xr = x.reshape(NW, ROWS_PER_W).astype(jnp.int32)
    return _make_gather()(xr, out_table)


# speedup vs baseline: 4.8325x; 1.9234x over previous
import functools
import jax
import jax.numpy as jnp
from jax import lax
from jax.experimental import pallas as pl
from jax.experimental.pallas import tpu as pltpu
from jax.experimental.pallas import tpu_sc as plsc

HID = 64
VOCAB = 64
ROWS = 4096
COLS = 200
B = ROWS * COLS
NPAIR = B // 2
NC = 2
NS = 16
NW = NC * NS
PAIRS_PER_W = NPAIR // NW
RPW = ROWS // NW
UNIT = COLS
NUNIT = PAIRS_PER_W // UNIT
G0, G1 = 128, UNIT - 128

NBUF = 3
DEPTH = 1


def _prep_body(x_ref, table_ref, w1_ref, b1_ref, w2_ref, b2_ref, gamma_ref,
               beta_ref, ptab_ref, xp_ref):
    h = table_ref[...]
    z = jnp.dot(h, w1_ref[...], preferred_element_type=jnp.float32) + b1_ref[...]
    z = jnp.maximum(z, 0.0)
    ff = jnp.dot(z, w2_ref[...], preferred_element_type=jnp.float32) + b2_ref[...]
    s = h + ff
    mu = jnp.mean(s, axis=-1, keepdims=True)
    var = jnp.mean((s - mu) * (s - mu), axis=-1, keepdims=True)
    shat = (s - mu) * lax.rsqrt(var + 1e-5)
    t = shat * gamma_ref[...] + beta_ref[...]

    b1t = jnp.broadcast_to(t[:, None, :], (VOCAB, VOCAB, HID))
    b2t = jnp.broadcast_to(t[None, :, :], (VOCAB, VOCAB, HID))
    ptab_ref[...] = jnp.concatenate(
        [b1t.reshape(VOCAB * VOCAB, HID), b2t.reshape(VOCAB * VOCAB, HID)],
        axis=1)

    xf = x_ref[...].astype(jnp.float32)
    rows = lax.broadcasted_iota(jnp.int32, (128, 64), 0)
    cols = lax.broadcasted_iota(jnp.int32, (128, 64), 1)
    sel_even = jnp.where(rows == 2 * cols, 1.0, 0.0).astype(jnp.float32)
    sel_odd = jnp.where(rows == 2 * cols + 1, 1.0, 0.0).astype(jnp.float32)
    xe = jnp.dot(xf, sel_even, preferred_element_type=jnp.float32)
    xo = jnp.dot(xf, sel_odd, preferred_element_type=jnp.float32)
    xp_ref[...] = (xe * 64.0 + xo).astype(jnp.int32)


def _prepare(x4, table, w1, b1, w2, b2, gamma, beta):
    return pl.pallas_call(
        _prep_body,
        out_shape=(
            jax.ShapeDtypeStruct((VOCAB * VOCAB, 2 * HID), jnp.float32),
            jax.ShapeDtypeStruct((B // 128, 64), jnp.int32),
        ),
    )(x4, table, w1, b1.reshape(1, -1), w2, b2.reshape(1, -1),
      gamma.reshape(1, -1), beta.reshape(1, -1))


@functools.cache
def _make_gather():
    mesh = plsc.VectorSubcoreMesh(core_axis_name="c", subcore_axis_name="s")

    @functools.partial(
        pl.kernel,
        out_type=jax.ShapeDtypeStruct((NPAIR, 2 * HID), jnp.float32),
        mesh=mesh,
        scratch_types=[
            pltpu.VMEM((PAIRS_PER_W,), jnp.int32),
            pltpu.VMEM((NBUF, UNIT, 2 * HID), jnp.float32),
            pltpu.SemaphoreType.DMA,
            pltpu.SemaphoreType.DMA,
        ],
        compiler_params=pltpu.CompilerParams(use_tc_tiling_on_sc=False),
    )
    def _gather(xp_hbm, ptab_hbm, out_hbm, idx_v, rows_v, gsem, ssem):
        wid = lax.axis_index("s") * NC + lax.axis_index("c")
        pltpu.sync_copy(xp_hbm.at[wid], idx_v)
        pair0 = wid * PAIRS_PER_W

        def issue_gathers(u, b):
            pltpu.async_copy(
                ptab_hbm.at[idx_v.at[pl.ds(u * UNIT, G0)]],
                rows_v.at[b, pl.ds(0, G0)], gsem)
            pltpu.async_copy(
                ptab_hbm.at[idx_v.at[pl.ds(u * UNIT + G0, G1)]],
                rows_v.at[b, pl.ds(G0, G1)], gsem)

        def wait_gathers(u, b):
            pltpu.make_async_copy(
                ptab_hbm.at[idx_v.at[pl.ds(u * UNIT, G0)]],
                rows_v.at[b, pl.ds(0, G0)], gsem).wait()
            pltpu.make_async_copy(
                ptab_hbm.at[idx_v.at[pl.ds(u * UNIT + G0, G1)]],
                rows_v.at[b, pl.ds(G0, G1)], gsem).wait()

        def out_slice(u):
            return out_hbm.at[pl.ds(pair0 + u * UNIT, UNIT)]

        def body(j, carry):
            @pl.when(j < NUNIT)
            def _():
                b = j % NBUF

                @pl.when(j >= NBUF)
                def _():
                    pltpu.make_async_copy(
                        rows_v.at[b], out_slice(j - NBUF), ssem).wait()

                issue_gathers(j, b)

            @pl.when(j >= DEPTH)
            def _():
                i = j - DEPTH
                bi = i % NBUF
                wait_gathers(i, bi)
                pltpu.async_copy(rows_v.at[bi], out_slice(i), ssem)

            return carry

        lax.fori_loop(0, NUNIT + DEPTH, body, 0)

        def drain(j, carry):
            pltpu.make_async_copy(
                rows_v.at[j % NBUF], out_slice(j), ssem).wait()
            return carry

        lax.fori_loop(NUNIT - NBUF, NUNIT, drain, 0)

    return _gather


def kernel(x, table, W1, b1, W2, b2, gamma, beta):
    x4 = x.reshape(B // 128, 128).astype(jnp.int32)
    ptab, xp = _prepare(x4, table, W1, b1, W2, b2, gamma, beta)
    xp_w = xp.reshape(NW, PAIRS_PER_W)
    out = _make_gather()(xp_w, ptab)
    return out.reshape(ROWS, COLS, HID)


# speedup vs baseline: 4.8401x; 1.0016x over previous
import functools
import jax
import jax.numpy as jnp
from jax import lax
from jax.experimental import pallas as pl
from jax.experimental.pallas import tpu as pltpu
from jax.experimental.pallas import tpu_sc as plsc

HID = 64
VOCAB = 64
ROWS = 4096
COLS = 200
B = ROWS * COLS
NPAIR = B // 2
NC = 2
NS = 16
NW = NC * NS
PAIRS_PER_W = NPAIR // NW
RPW = ROWS // NW
UNIT = COLS
NUNIT = PAIRS_PER_W // UNIT
G0, G1 = 128, UNIT - 128

NBUF = 3
DEPTH = 1


def _prep_body(x_ref, table_ref, w1_ref, b1_ref, w2_ref, b2_ref, gamma_ref,
               beta_ref, ptab_ref, xp_ref):
    h = table_ref[...]
    z = jnp.dot(h, w1_ref[...], preferred_element_type=jnp.float32) + b1_ref[...]
    z = jnp.maximum(z, 0.0)
    ff = jnp.dot(z, w2_ref[...], preferred_element_type=jnp.float32) + b2_ref[...]
    s = h + ff
    mu = jnp.mean(s, axis=-1, keepdims=True)
    var = jnp.mean((s - mu) * (s - mu), axis=-1, keepdims=True)
    shat = (s - mu) * lax.rsqrt(var + 1e-5)
    t = shat * gamma_ref[...] + beta_ref[...]

    b1t = jnp.broadcast_to(t[:, None, :], (VOCAB, VOCAB, HID))
    b2t = jnp.broadcast_to(t[None, :, :], (VOCAB, VOCAB, HID))
    ptab_ref[...] = jnp.concatenate(
        [b1t.reshape(VOCAB * VOCAB, HID), b2t.reshape(VOCAB * VOCAB, HID)],
        axis=1)

    xf = x_ref[...].astype(jnp.float32)
    rows = lax.broadcasted_iota(jnp.int32, (128, 64), 0)
    cols = lax.broadcasted_iota(jnp.int32, (128, 64), 1)
    sel_even = jnp.where(rows == 2 * cols, 1.0, 0.0).astype(jnp.float32)
    sel_odd = jnp.where(rows == 2 * cols + 1, 1.0, 0.0).astype(jnp.float32)
    xe = jnp.dot(xf, sel_even, preferred_element_type=jnp.float32)
    xo = jnp.dot(xf, sel_odd, preferred_element_type=jnp.float32)
    xp_ref[...] = (xe * 64.0 + xo).astype(jnp.int32)


def _prepare(x4, table, w1, b1, w2, b2, gamma, beta):
    return pl.pallas_call(
        _prep_body,
        out_shape=(
            jax.ShapeDtypeStruct((VOCAB * VOCAB, 2 * HID), jnp.float32),
            jax.ShapeDtypeStruct((B // 128, 64), jnp.int32),
        ),
    )(x4, table, w1, b1.reshape(1, -1), w2, b2.reshape(1, -1),
      gamma.reshape(1, -1), beta.reshape(1, -1))


@functools.cache
def _make_gather():
    mesh = plsc.VectorSubcoreMesh(core_axis_name="c", subcore_axis_name="s")

    @functools.partial(
        pl.kernel,
        out_type=jax.ShapeDtypeStruct((NPAIR, 2 * HID), jnp.float32),
        mesh=mesh,
        scratch_types=[
            pltpu.VMEM((PAIRS_PER_W,), jnp.int32),
            pltpu.VMEM((NBUF, UNIT, 2 * HID), jnp.float32),
            pltpu.SemaphoreType.DMA,
            pltpu.SemaphoreType.DMA,
        ],
        compiler_params=pltpu.CompilerParams(use_tc_tiling_on_sc=True),
    )
    def _gather(xp_hbm, ptab_hbm, out_hbm, idx_v, rows_v, gsem, ssem):
        wid = lax.axis_index("s") * NC + lax.axis_index("c")
        pltpu.sync_copy(xp_hbm.at[wid], idx_v)
        pair0 = wid * PAIRS_PER_W

        def issue_gathers(u, b):
            pltpu.async_copy(
                ptab_hbm.at[idx_v.at[pl.ds(u * UNIT, G0)]],
                rows_v.at[b, pl.ds(0, G0)], gsem)
            pltpu.async_copy(
                ptab_hbm.at[idx_v.at[pl.ds(u * UNIT + G0, G1)]],
                rows_v.at[b, pl.ds(G0, G1)], gsem)

        def wait_gathers(u, b):
            pltpu.make_async_copy(
                ptab_hbm.at[idx_v.at[pl.ds(u * UNIT, G0)]],
                rows_v.at[b, pl.ds(0, G0)], gsem).wait()
            pltpu.make_async_copy(
                ptab_hbm.at[idx_v.at[pl.ds(u * UNIT + G0, G1)]],
                rows_v.at[b, pl.ds(G0, G1)], gsem).wait()

        def out_slice(u):
            return out_hbm.at[pl.ds(pair0 + u * UNIT, UNIT)]

        def body(j, carry):
            @pl.when(j < NUNIT)
            def _():
                b = j % NBUF

                @pl.when(j >= NBUF)
                def _():
                    pltpu.make_async_copy(
                        rows_v.at[b], out_slice(j - NBUF), ssem).wait()

                issue_gathers(j, b)

            @pl.when(j >= DEPTH)
            def _():
                i = j - DEPTH
                bi = i % NBUF
                wait_gathers(i, bi)
                pltpu.async_copy(rows_v.at[bi], out_slice(i), ssem)

            return carry

        lax.fori_loop(0, NUNIT + DEPTH, body, 0)

        def drain(j, carry):
            pltpu.make_async_copy(
                rows_v.at[j % NBUF], out_slice(j), ssem).wait()
            return carry

        lax.fori_loop(NUNIT - NBUF, NUNIT, drain, 0)

    return _gather


def kernel(x, table, W1, b1, W2, b2, gamma, beta):
    x4 = x.reshape(B // 128, 128).astype(jnp.int32)
    ptab, xp = _prepare(x4, table, W1, b1, W2, b2, gamma, beta)
    xp_w = xp.reshape(NW, PAIRS_PER_W)
    out = _make_gather()(xp_w, ptab)
    return out.reshape(ROWS, COLS, HID)


# speedup vs baseline: 6.3403x; 1.3099x over previous
import functools
import jax
import jax.numpy as jnp
from jax import lax
from jax.experimental import pallas as pl
from jax.experimental.pallas import tpu as pltpu
from jax.experimental.pallas import tpu_sc as plsc

HID = 64
VOCAB = 64
ROWS = 4096
COLS = 200
B = ROWS * COLS
NPAIR = B // 2
PW = 4 * HID
NC = 2
NS = 16
NW = NC * NS
PAIRS_PER_W = NPAIR // NW
UNIT = 128
NUNIT = PAIRS_PER_W // UNIT

NBUF = 3
DEPTH = 1


def _prep_body(x_ref, table_ref, w1_ref, b1_ref, w2_ref, b2_ref, gamma_ref,
               beta_ref, ptab_ref, xp_ref):
    h = table_ref[...]
    z = jnp.dot(h, w1_ref[...], preferred_element_type=jnp.float32) + b1_ref[...]
    z = jnp.maximum(z, 0.0)
    ff = jnp.dot(z, w2_ref[...], preferred_element_type=jnp.float32) + b2_ref[...]
    s = h + ff
    mu = jnp.mean(s, axis=-1, keepdims=True)
    var = jnp.mean((s - mu) * (s - mu), axis=-1, keepdims=True)
    shat = (s - mu) * lax.rsqrt(var + 1e-5)
    t = shat * gamma_ref[...] + beta_ref[...]

    b1t = jnp.broadcast_to(t[:, None, :], (VOCAB, VOCAB, HID))
    b2t = jnp.broadcast_to(t[None, :, :], (VOCAB, VOCAB, HID))
    zpad = jnp.zeros((VOCAB * VOCAB, HID), jnp.float32)
    ptab_ref[...] = jnp.concatenate(
        [b1t.reshape(VOCAB * VOCAB, HID), zpad,
         b2t.reshape(VOCAB * VOCAB, HID), zpad], axis=1)

    xf = x_ref[...].astype(jnp.float32)
    rows = lax.broadcasted_iota(jnp.int32, (128, 64), 0)
    cols = lax.broadcasted_iota(jnp.int32, (128, 64), 1)
    sel_even = jnp.where(rows == 2 * cols, 1.0, 0.0).astype(jnp.float32)
    sel_odd = jnp.where(rows == 2 * cols + 1, 1.0, 0.0).astype(jnp.float32)
    xe = jnp.dot(xf, sel_even, preferred_element_type=jnp.float32)
    xo = jnp.dot(xf, sel_odd, preferred_element_type=jnp.float32)
    xp_ref[...] = (xe * 64.0 + xo).astype(jnp.int32)


def _prepare(x4, table, w1, b1, w2, b2, gamma, beta):
    return pl.pallas_call(
        _prep_body,
        out_shape=(
            jax.ShapeDtypeStruct((VOCAB * VOCAB, PW), jnp.float32),
            jax.ShapeDtypeStruct((B // 128, 64), jnp.int32),
        ),
    )(x4, table, w1, b1.reshape(1, -1), w2, b2.reshape(1, -1),
      gamma.reshape(1, -1), beta.reshape(1, -1))


@functools.cache
def _make_gather():
    mesh = plsc.VectorSubcoreMesh(core_axis_name="c", subcore_axis_name="s")

    @functools.partial(
        pl.kernel,
        out_type=jax.ShapeDtypeStruct((NPAIR, PW), jnp.float32),
        mesh=mesh,
        scratch_types=[
            pltpu.VMEM((PAIRS_PER_W,), jnp.int32),
            pltpu.VMEM((NBUF, UNIT, PW), jnp.float32),
            pltpu.SemaphoreType.DMA,
            pltpu.SemaphoreType.DMA,
        ],
        compiler_params=pltpu.CompilerParams(use_tc_tiling_on_sc=False),
    )
    def _gather(xp_hbm, ptab_hbm, out_hbm, idx_v, rows_v, gsem, ssem):
        wid = lax.axis_index("s") * NC + lax.axis_index("c")
        pltpu.sync_copy(xp_hbm.at[wid], idx_v)
        pair0 = wid * PAIRS_PER_W

        def gather_copy(u, b):
            return pltpu.make_async_copy(
                ptab_hbm.at[idx_v.at[pl.ds(u * UNIT, UNIT)]],
                rows_v.at[b], gsem)

        def scatter_copy(u, b):
            return pltpu.make_async_copy(
                rows_v.at[b], out_hbm.at[pl.ds(pair0 + u * UNIT, UNIT)],
                ssem)

        def body(j, carry):
            @pl.when(j < NUNIT)
            def _():
                b = j % NBUF

                @pl.when(j >= NBUF)
                def _():
                    scatter_copy(j - NBUF, b).wait()

                gather_copy(j, b).start()

            @pl.when(j >= DEPTH)
            def _():
                i = j - DEPTH
                bi = i % NBUF
                gather_copy(i, bi).wait()
                scatter_copy(i, bi).start()

            return carry

        lax.fori_loop(0, NUNIT + DEPTH, body, 0)

        def drain(j, carry):
            scatter_copy(j, j % NBUF).wait()
            return carry

        lax.fori_loop(NUNIT - NBUF, NUNIT, drain, 0)

    return _gather


def kernel(x, table, W1, b1, W2, b2, gamma, beta):
    x4 = x.reshape(B // 128, 128).astype(jnp.int32)
    ptab, xp = _prepare(x4, table, W1, b1, W2, b2, gamma, beta)
    xp_w = xp.reshape(NW, PAIRS_PER_W)
    out = _make_gather()(xp_w, ptab)
    return out.reshape(B, 2 * HID)[:, :HID].reshape(ROWS, COLS, HID)


# speedup vs baseline: 8.5907x; 1.3549x over previous
import functools
import jax
import jax.numpy as jnp
from jax import lax
from jax.experimental import pallas as pl
from jax.experimental.pallas import tpu as pltpu
from jax.experimental.pallas import tpu_sc as plsc

HID = 64
VOCAB = 64
ROWS = 4096
COLS = 200
B = ROWS * COLS
NPAIR = B // 2
PW = 4 * HID
NC = 2
NS = 16
NW = NC * NS
PAIRS_PER_W = NPAIR // NW
UNIT = 128
NUNIT = PAIRS_PER_W // UNIT

NBUF = 3
DEPTH = 1


def _prep_body(x_ref, table_ref, w1_ref, b1_ref, w2_ref, b2_ref, gamma_ref,
               beta_ref, ptab_ref, xp_ref):
    h = table_ref[...]
    z = jnp.dot(h, w1_ref[...], preferred_element_type=jnp.float32) + b1_ref[...]
    z = jnp.maximum(z, 0.0)
    ff = jnp.dot(z, w2_ref[...], preferred_element_type=jnp.float32) + b2_ref[...]
    s = h + ff
    mu = jnp.mean(s, axis=-1, keepdims=True)
    var = jnp.mean((s - mu) * (s - mu), axis=-1, keepdims=True)
    shat = (s - mu) * lax.rsqrt(var + 1e-5)
    t = shat * gamma_ref[...] + beta_ref[...]

    b1t = jnp.broadcast_to(t[:, None, :], (VOCAB, VOCAB, HID))
    b2t = jnp.broadcast_to(t[None, :, :], (VOCAB, VOCAB, HID))
    ptab_ref[...] = jnp.concatenate(
        [b1t.reshape(VOCAB * VOCAB, HID),
         b2t.reshape(VOCAB * VOCAB, HID)], axis=1)

    xf = x_ref[...].astype(jnp.float32)
    rows = lax.broadcasted_iota(jnp.int32, (128, 64), 0)
    cols = lax.broadcasted_iota(jnp.int32, (128, 64), 1)
    sel_even = jnp.where(rows == 2 * cols, 1.0, 0.0).astype(jnp.float32)
    sel_odd = jnp.where(rows == 2 * cols + 1, 1.0, 0.0).astype(jnp.float32)
    xe = jnp.dot(xf, sel_even, preferred_element_type=jnp.float32)
    xo = jnp.dot(xf, sel_odd, preferred_element_type=jnp.float32)
    xp_ref[...] = (xe * 64.0 + xo).astype(jnp.int32)


def _prepare(x4, table, w1, b1, w2, b2, gamma, beta):
    return pl.pallas_call(
        _prep_body,
        out_shape=(
            jax.ShapeDtypeStruct((VOCAB * VOCAB, 2 * HID), jnp.float32),
            jax.ShapeDtypeStruct((B // 128, 64), jnp.int32),
        ),
    )(x4, table, w1, b1.reshape(1, -1), w2, b2.reshape(1, -1),
      gamma.reshape(1, -1), beta.reshape(1, -1))


@functools.cache
def _make_gather():
    mesh = plsc.VectorSubcoreMesh(core_axis_name="c", subcore_axis_name="s")

    @functools.partial(
        pl.kernel,
        out_type=jax.ShapeDtypeStruct((NPAIR, PW), jnp.float32),
        mesh=mesh,
        scratch_types=[
            pltpu.VMEM((PAIRS_PER_W,), jnp.int32),
            pltpu.VMEM((NBUF, UNIT, 2 * HID), jnp.float32),
            pltpu.SemaphoreType.DMA,
            pltpu.SemaphoreType.DMA,
        ],
        compiler_params=pltpu.CompilerParams(use_tc_tiling_on_sc=False),
    )
    def _gather(xp_hbm, ptab_hbm, out_hbm, idx_v, rows_v, gsem, ssem):
        wid = lax.axis_index("s") * NC + lax.axis_index("c")
        pltpu.sync_copy(xp_hbm.at[wid], idx_v)
        pair0 = wid * PAIRS_PER_W

        def gather_copy(u, b):
            return pltpu.make_async_copy(
                ptab_hbm.at[idx_v.at[pl.ds(u * UNIT, UNIT)]],
                rows_v.at[b], gsem)

        def scatter_copies(u, b):
            dst = out_hbm.at[pl.ds(pair0 + u * UNIT, UNIT)]
            return (
                pltpu.make_async_copy(
                    rows_v.at[b, slice(None), pl.ds(0, HID)],
                    dst.at[slice(None), pl.ds(0, HID)], ssem),
                pltpu.make_async_copy(
                    rows_v.at[b, slice(None), pl.ds(HID, HID)],
                    dst.at[slice(None), pl.ds(2 * HID, HID)], ssem),
            )

        def body(j, carry):
            @pl.when(j < NUNIT)
            def _():
                b = j % NBUF

                @pl.when(j >= NBUF)
                def _():
                    for c in scatter_copies(j - NBUF, b):
                        c.wait()

                gather_copy(j, b).start()

            @pl.when(j >= DEPTH)
            def _():
                i = j - DEPTH
                bi = i % NBUF
                gather_copy(i, bi).wait()
                for c in scatter_copies(i, bi):
                    c.start()

            return carry

        lax.fori_loop(0, NUNIT + DEPTH, body, 0)

        def drain(j, carry):
            for c in scatter_copies(j, j % NBUF):
                c.wait()
            return carry

        lax.fori_loop(NUNIT - NBUF, NUNIT, drain, 0)

    return _gather


def kernel(x, table, W1, b1, W2, b2, gamma, beta):
    x4 = x.reshape(B // 128, 128).astype(jnp.int32)
    ptab, xp = _prepare(x4, table, W1, b1, W2, b2, gamma, beta)
    xp_w = xp.reshape(NW, PAIRS_PER_W)
    out = _make_gather()(xp_w, ptab)
    return out.reshape(B, 2 * HID)[:, :HID].reshape(ROWS, COLS, HID)
